# baseline (speedup 1.0000x reference)
import functools

import jax
import jax.numpy as jnp
from jax import lax
from jax.experimental import pallas as pl
from jax.experimental.pallas import tpu as pltpu
from jax.experimental.pallas import tpu_sc as plsc

N = 10000
F = 128
H = 256
HALF = 128
E = 320000
NS = 16
EPT = E // NS
K = 80
NCH = EPT // K
RPT = 624
RPT_LAST = N - 15 * RPT

BM = 2000
DM = 400


NB = 4


def _segsum_body(table_hbm, src_hbm, dst_hbm, zeros_hbm, out_hbm, *scr):
    accum = scr[6 * NB]
    bufs = tuple(scr[6 * b:6 * b + 6] for b in range(NB))
    c = lax.axis_index("c")
    s = lax.axis_index("s")

    @pl.when(s < NS - 1)
    def _():
        pltpu.sync_copy(zeros_hbm.at[pl.ds(0, RPT)], accum.at[pl.ds(s * RPT, RPT)])

    @pl.when(s == NS - 1)
    def _():
        pltpu.sync_copy(zeros_hbm, accum.at[pl.ds((NS - 1) * RPT, RPT_LAST)])

    plsc.subcore_barrier()

    ebase = s * EPT
    off = c * N

    def A(b):
        _, dstb, rows, _, _, sems = bufs[b]
        pltpu.make_async_copy(rows, accum.at[dstb], sems).wait()

    def B(b, i):
        srcb, dstb, _, semi, _, _ = bufs[b]
        e0 = ebase + i * K
        pltpu.async_copy(src_hbm.at[pl.ds(e0, K)], srcb, semi)
        pltpu.async_copy(dst_hbm.at[pl.ds(e0, K)], dstb, semi)

    def C(b):
        srcb, dstb, rows, semi, semg, _ = bufs[b]
        pltpu.make_async_copy(src_hbm.at[pl.ds(0, K)], srcb, semi).wait()
        pltpu.make_async_copy(dst_hbm.at[pl.ds(0, K)], dstb, semi).wait()
        for j in range(K // 16):
            sl = pl.ds(j * 16, 16)
            srcb[sl] = srcb[sl] + off
        pltpu.async_copy(table_hbm.at[srcb], rows, semg)

    def D(b):
        srcb, dstb, rows, _, semg, sems = bufs[b]
        pltpu.make_async_copy(table_hbm.at[srcb], rows, semg).wait()
        pltpu.async_copy(rows, accum.at[dstb], sems, add=True)

    def chunk_step(i, i_static):
        if i_static >= 2:
            A((i_static - 2) % NB)
        if i_static + 2 < NCH:
            B((i_static + 2) % NB, i + 2)
        if i_static + 1 < NCH:
            C((i_static + 1) % NB)
        D(i_static % NB)

    B(0, 0)
    B(1, 1)
    C(0)
    for i in range(4):
        chunk_step(i, i)

    def group(g, _):
        i0 = 4 + 4 * g
        for k in range(4):
            chunk_step(i0 + k, 4 + k)
        return 0

    lax.fori_loop(0, (NCH - 10) // 4, group, 0)
    for i in range(NCH - 6, NCH):
        chunk_step(i, i)
    A((NCH - 2) % NB)
    A((NCH - 1) % NB)
    plsc.subcore_barrier()

    @pl.when(s < NS - 1)
    def _():
        pltpu.sync_copy(accum.at[pl.ds(s * RPT, RPT)],
                        out_hbm.at[pl.ds(c * N + s * RPT, RPT)])

    @pl.when(s == NS - 1)
    def _():
        pltpu.sync_copy(accum.at[pl.ds((NS - 1) * RPT, RPT_LAST)],
                        out_hbm.at[pl.ds(c * N + (NS - 1) * RPT, RPT_LAST)])


_segsum = pl.kernel(
    _segsum_body,
    out_type=jax.ShapeDtypeStruct((2 * N, HALF), jnp.float32),
    mesh=plsc.VectorSubcoreMesh(core_axis_name="c", subcore_axis_name="s"),
    scratch_types=(
        [t for _ in range(NB) for t in (
            pltpu.VMEM((K,), jnp.int32),
            pltpu.VMEM((K,), jnp.int32),
            pltpu.VMEM((K, HALF), jnp.float32),
            pltpu.SemaphoreType.DMA,
            pltpu.SemaphoreType.DMA,
            pltpu.SemaphoreType.DMA,
        )] + [pltpu.VMEM_SHARED((N, HALF), jnp.float32)]
    ),
)


def _in_mm_body(x_ref, w1_ref, w2_ref, o1_ref, o2_ref):
    x = x_ref[...]
    o1_ref[0] = jnp.dot(x, w1_ref[...], preferred_element_type=jnp.float32)
    o2_ref[0] = jnp.dot(x, w2_ref[...], preferred_element_type=jnp.float32)


def _input_matmuls(x, w1, w2):
    grid = (2, N // BM)
    out = pl.pallas_call(
        _in_mm_body,
        grid=grid,
        in_specs=[
            pl.BlockSpec((BM, F), lambda c, r: (r, 0)),
            pl.BlockSpec((F, HALF), lambda c, r: (0, c)),
            pl.BlockSpec((F, HALF), lambda c, r: (0, c)),
        ],
        out_specs=[
            pl.BlockSpec((1, BM, HALF), lambda c, r: (c, r, 0)),
            pl.BlockSpec((1, BM, HALF), lambda c, r: (c, r, 0)),
        ],
        out_shape=[
            jax.ShapeDtypeStruct((2, N, HALF), jnp.float32),
            jax.ShapeDtypeStruct((2, N, HALF), jnp.float32),
        ],
    )(x, w1, w2)
    return out


def _hid_mm_body(a0_ref, a1_ref, b0_ref, b1_ref, w12_ref, w1ls_ref, w2ls_ref,
                 p12_ref, p1ls_ref, p2ls_ref):
    a0 = jnp.maximum(a0_ref[0], 0.0)
    a1 = jnp.maximum(a1_ref[0], 0.0)
    b0 = jnp.maximum(b0_ref[0], 0.0)
    b1 = jnp.maximum(b1_ref[0], 0.0)

    def mm2(x0, x1, w_ref):
        return (jnp.dot(x0, w_ref[0:HALF, :], preferred_element_type=jnp.float32)
                + jnp.dot(x1, w_ref[HALF:H, :], preferred_element_type=jnp.float32))

    p12_ref[0] = mm2(a0, a1, w12_ref)
    p1ls_ref[0] = mm2(a0, a1, w1ls_ref)
    p2ls_ref[0] = mm2(b0, b1, w2ls_ref)


def _hidden_matmuls(h11, h21, w12, w1ls, w2ls):
    grid = (2, N // BM)
    stacked = lambda k: pl.BlockSpec((1, BM, HALF), lambda c, r, k=k: (k, r, 0))
    return pl.pallas_call(
        _hid_mm_body,
        grid=grid,
        in_specs=[
            stacked(0), stacked(1), stacked(0), stacked(1),
            pl.BlockSpec((H, HALF), lambda c, r: (0, c)),
            pl.BlockSpec((H, HALF), lambda c, r: (0, c)),
            pl.BlockSpec((H, HALF), lambda c, r: (0, c)),
        ],
        out_specs=[
            pl.BlockSpec((1, BM, HALF), lambda c, r: (c, r, 0)),
            pl.BlockSpec((1, BM, HALF), lambda c, r: (c, r, 0)),
            pl.BlockSpec((1, BM, HALF), lambda c, r: (c, r, 0)),
        ],
        out_shape=[
            jax.ShapeDtypeStruct((2, N, HALF), jnp.float32),
            jax.ShapeDtypeStruct((2, N, HALF), jnp.float32),
            jax.ShapeDtypeStruct((2, N, HALF), jnp.float32),
        ],
    )(h11, h11, h21, h21, w12, w1ls, w2ls)


def _fusion_body(h11_ref, h12_ref, ls1_ref, ls2_ref, eps_ref, z_ref):
    z_mean = jnp.maximum(h11_ref[0], 0.0) + h12_ref[0]
    z_log_std = ls1_ref[0] + ls2_ref[0]
    z = z_mean + eps_ref[...] * jnp.exp(z_log_std)
    z_ref[...] = z.astype(jnp.bfloat16)


def _fusion(h11, h12, ls1, ls2, eps):
    grid = (2, N // BM)
    stk = pl.BlockSpec((1, BM, HALF), lambda c, r: (c, r, 0))
    return pl.pallas_call(
        _fusion_body,
        grid=grid,
        in_specs=[stk, stk, stk, stk,
                  pl.BlockSpec((BM, HALF), lambda c, r: (r, c))],
        out_specs=pl.BlockSpec((BM, HALF), lambda c, r: (r, c)),
        out_shape=jax.ShapeDtypeStruct((N, H), jnp.bfloat16),
    )(h11, h12, ls1, ls2, eps)


def _decoder_body(zl_ref, zr_ref, o_ref):
    o_ref[...] = lax.dot_general(
        zl_ref[...], zr_ref[...], (((1,), (1,)), ((), ())),
        preferred_element_type=jnp.float32)


def _decoder(z):
    return pl.pallas_call(
        _decoder_body,
        grid=(N // DM,),
        in_specs=[
            pl.BlockSpec((DM, H), lambda r: (r, 0)),
            pl.BlockSpec((N, H), lambda r: (0, 0)),
        ],
        out_specs=pl.BlockSpec((DM, N), lambda r: (r, 0)),
        out_shape=jax.ShapeDtypeStruct((N, N), jnp.float32),
    )(z, z)


def kernel(features, edge_index1, edge_index2, W1_1, W1_2, W1_ls, W2_1, W2_2, W2_ls):
    src1, dst1 = edge_index1[0], edge_index1[1]
    src2, dst2 = edge_index2[0], edge_index2[1]
    zeros = jnp.zeros((RPT_LAST, HALF), jnp.float32)

    xw1, xw2 = _input_matmuls(features, W1_1, W2_1)
    h11 = _segsum(xw1.reshape(2 * N, HALF), src1, dst1, zeros)
    h21 = _segsum(xw2.reshape(2 * N, HALF), src2, dst2, zeros)

    p12, p1ls, p2ls = _hidden_matmuls(
        h11.reshape(2, N, HALF), h21.reshape(2, N, HALF), W1_2, W1_ls, W2_ls)
    h12 = _segsum(p12.reshape(2 * N, HALF), src1, dst1, zeros)
    ls1 = _segsum(p1ls.reshape(2 * N, HALF), src1, dst1, zeros)
    ls2 = _segsum(p2ls.reshape(2 * N, HALF), src2, dst2, zeros)

    eps = jax.random.normal(jax.random.key(42), (N, H), dtype=jnp.float32)
    z = _fusion(h11.reshape(2, N, HALF), h12.reshape(2, N, HALF),
                ls1.reshape(2, N, HALF), ls2.reshape(2, N, HALF), eps)

    rec = _decoder(z).reshape(-1)
    return (rec, rec)


# speedup vs baseline: 1.0332x; 1.0332x over previous
import functools

import jax
import jax.numpy as jnp
from jax import lax
from jax.experimental import pallas as pl
from jax.experimental.pallas import tpu as pltpu
from jax.experimental.pallas import tpu_sc as plsc

N = 10000
F = 128
H = 256
HALF = 128
E = 320000
NS = 16
EPT = E // NS
K = 80
NCH = EPT // K
RPT = 624
RPT_LAST = N - 15 * RPT

BM = 2000
DM = 400


NB = 4


def _accum_zero(zeros_hbm, accum, s):
    @pl.when(s < NS - 1)
    def _():
        pltpu.sync_copy(zeros_hbm.at[pl.ds(0, RPT)], accum.at[pl.ds(s * RPT, RPT)])

    @pl.when(s == NS - 1)
    def _():
        pltpu.sync_copy(zeros_hbm, accum.at[pl.ds((NS - 1) * RPT, RPT_LAST)])


def _writeout(accum, out_hbm, c, s):
    @pl.when(s < NS - 1)
    def _():
        pltpu.sync_copy(accum.at[pl.ds(s * RPT, RPT)],
                        out_hbm.at[pl.ds(c * N + s * RPT, RPT)])

    @pl.when(s == NS - 1)
    def _():
        pltpu.sync_copy(accum.at[pl.ds((NS - 1) * RPT, RPT_LAST)],
                        out_hbm.at[pl.ds(c * N + (NS - 1) * RPT, RPT_LAST)])


def _seg_phase(table_hbm, src_hbm, dst_hbm, bufs, accum, c, s):
    ebase = s * EPT
    off = c * N

    def A(b):
        _, dstb, rows, _, _, sems = bufs[b]
        pltpu.make_async_copy(rows, accum.at[dstb], sems).wait()

    def B(b, i):
        srcb, dstb, _, semi, _, _ = bufs[b]
        e0 = ebase + i * K
        pltpu.async_copy(src_hbm.at[pl.ds(e0, K)], srcb, semi)
        pltpu.async_copy(dst_hbm.at[pl.ds(e0, K)], dstb, semi)

    def C(b):
        srcb, dstb, rows, semi, semg, _ = bufs[b]
        pltpu.make_async_copy(src_hbm.at[pl.ds(0, K)], srcb, semi).wait()
        pltpu.make_async_copy(dst_hbm.at[pl.ds(0, K)], dstb, semi).wait()
        for j in range(K // 16):
            sl = pl.ds(j * 16, 16)
            srcb[sl] = srcb[sl] + off
        pltpu.async_copy(table_hbm.at[srcb], rows, semg)

    def D(b):
        srcb, dstb, rows, _, semg, sems = bufs[b]
        pltpu.make_async_copy(table_hbm.at[srcb], rows, semg).wait()
        pltpu.async_copy(rows, accum.at[dstb], sems, add=True)

    def chunk_step(i, i_static):
        if i_static >= 2:
            A((i_static - 2) % NB)
        if i_static + 2 < NCH:
            B((i_static + 2) % NB, i + 2)
        if i_static + 1 < NCH:
            C((i_static + 1) % NB)
        D(i_static % NB)

    B(0, 0)
    B(1, 1)
    C(0)
    for i in range(4):
        chunk_step(i, i)

    def group(g, _):
        i0 = 4 + 4 * g
        for k in range(4):
            chunk_step(i0 + k, 4 + k)
        return 0

    lax.fori_loop(0, (NCH - 10) // 4, group, 0)
    for i in range(NCH - 6, NCH):
        chunk_step(i, i)
    A((NCH - 2) % NB)
    A((NCH - 1) % NB)


def _enc1_body(xw1, src1, dst1, xw2, src2, dst2, zeros_hbm, h11_out, h21_out, *scr):
    accum = scr[6 * NB]
    bufs = tuple(scr[6 * b:6 * b + 6] for b in range(NB))
    c = lax.axis_index("c")
    s = lax.axis_index("s")

    _accum_zero(zeros_hbm, accum, s)
    plsc.subcore_barrier()
    _seg_phase(xw1, src1, dst1, bufs, accum, c, s)
    plsc.subcore_barrier()
    _writeout(accum, h11_out, c, s)
    _accum_zero(zeros_hbm, accum, s)
    plsc.subcore_barrier()
    _seg_phase(xw2, src2, dst2, bufs, accum, c, s)
    plsc.subcore_barrier()
    _writeout(accum, h21_out, c, s)


def _enc2_body(p12, src1, dst1, p1ls, p2ls, src2, dst2, zeros_hbm,
               h12_out, ls_out, *scr):
    accum = scr[6 * NB]
    bufs = tuple(scr[6 * b:6 * b + 6] for b in range(NB))
    c = lax.axis_index("c")
    s = lax.axis_index("s")

    _accum_zero(zeros_hbm, accum, s)
    plsc.subcore_barrier()
    _seg_phase(p12, src1, dst1, bufs, accum, c, s)
    plsc.subcore_barrier()
    _writeout(accum, h12_out, c, s)
    _accum_zero(zeros_hbm, accum, s)
    plsc.subcore_barrier()
    _seg_phase(p1ls, src1, dst1, bufs, accum, c, s)
    _seg_phase(p2ls, src2, dst2, bufs, accum, c, s)
    plsc.subcore_barrier()
    _writeout(accum, ls_out, c, s)


_SEG_SCRATCH = (
    [t for _ in range(NB) for t in (
        pltpu.VMEM((K,), jnp.int32),
        pltpu.VMEM((K,), jnp.int32),
        pltpu.VMEM((K, HALF), jnp.float32),
        pltpu.SemaphoreType.DMA,
        pltpu.SemaphoreType.DMA,
        pltpu.SemaphoreType.DMA,
    )] + [pltpu.VMEM_SHARED((N, HALF), jnp.float32)]
)

_SEG_OUT = jax.ShapeDtypeStruct((2 * N, HALF), jnp.float32)

_enc1 = pl.kernel(
    _enc1_body,
    out_type=(_SEG_OUT, _SEG_OUT),
    mesh=plsc.VectorSubcoreMesh(core_axis_name="c", subcore_axis_name="s"),
    scratch_types=list(_SEG_SCRATCH),
)

_enc2 = pl.kernel(
    _enc2_body,
    out_type=(_SEG_OUT, _SEG_OUT),
    mesh=plsc.VectorSubcoreMesh(core_axis_name="c", subcore_axis_name="s"),
    scratch_types=list(_SEG_SCRATCH),
)


def _in_mm_body(x_ref, w1_ref, w2_ref, o1_ref, o2_ref):
    x = x_ref[...]
    o1_ref[0] = jnp.dot(x, w1_ref[...], preferred_element_type=jnp.float32)
    o2_ref[0] = jnp.dot(x, w2_ref[...], preferred_element_type=jnp.float32)


def _input_matmuls(x, w1, w2):
    grid = (2, N // BM)
    out = pl.pallas_call(
        _in_mm_body,
        grid=grid,
        in_specs=[
            pl.BlockSpec((BM, F), lambda c, r: (r, 0)),
            pl.BlockSpec((F, HALF), lambda c, r: (0, c)),
            pl.BlockSpec((F, HALF), lambda c, r: (0, c)),
        ],
        out_specs=[
            pl.BlockSpec((1, BM, HALF), lambda c, r: (c, r, 0)),
            pl.BlockSpec((1, BM, HALF), lambda c, r: (c, r, 0)),
        ],
        out_shape=[
            jax.ShapeDtypeStruct((2, N, HALF), jnp.float32),
            jax.ShapeDtypeStruct((2, N, HALF), jnp.float32),
        ],
    )(x, w1, w2)
    return out


def _hid_mm_body(a0_ref, a1_ref, b0_ref, b1_ref, w12_ref, w1ls_ref, w2ls_ref,
                 p12_ref, p1ls_ref, p2ls_ref):
    a0 = jnp.maximum(a0_ref[0], 0.0)
    a1 = jnp.maximum(a1_ref[0], 0.0)
    b0 = jnp.maximum(b0_ref[0], 0.0)
    b1 = jnp.maximum(b1_ref[0], 0.0)

    def mm2(x0, x1, w_ref):
        return (jnp.dot(x0, w_ref[0:HALF, :], preferred_element_type=jnp.float32)
                + jnp.dot(x1, w_ref[HALF:H, :], preferred_element_type=jnp.float32))

    p12_ref[0] = mm2(a0, a1, w12_ref)
    p1ls_ref[0] = mm2(a0, a1, w1ls_ref)
    p2ls_ref[0] = mm2(b0, b1, w2ls_ref)


def _hidden_matmuls(h11, h21, w12, w1ls, w2ls):
    grid = (2, N // BM)
    stacked = lambda k: pl.BlockSpec((1, BM, HALF), lambda c, r, k=k: (k, r, 0))
    return pl.pallas_call(
        _hid_mm_body,
        grid=grid,
        in_specs=[
            stacked(0), stacked(1), stacked(0), stacked(1),
            pl.BlockSpec((H, HALF), lambda c, r: (0, c)),
            pl.BlockSpec((H, HALF), lambda c, r: (0, c)),
            pl.BlockSpec((H, HALF), lambda c, r: (0, c)),
        ],
        out_specs=[
            pl.BlockSpec((1, BM, HALF), lambda c, r: (c, r, 0)),
            pl.BlockSpec((1, BM, HALF), lambda c, r: (c, r, 0)),
            pl.BlockSpec((1, BM, HALF), lambda c, r: (c, r, 0)),
        ],
        out_shape=[
            jax.ShapeDtypeStruct((2, N, HALF), jnp.float32),
            jax.ShapeDtypeStruct((2, N, HALF), jnp.float32),
            jax.ShapeDtypeStruct((2, N, HALF), jnp.float32),
        ],
    )(h11, h11, h21, h21, w12, w1ls, w2ls)


def _fusion_body(h11_ref, h12_ref, ls_ref, eps_ref, z_ref):
    z_mean = jnp.maximum(h11_ref[0], 0.0) + h12_ref[0]
    z = z_mean + eps_ref[...] * jnp.exp(ls_ref[0])
    z_ref[...] = z.astype(jnp.bfloat16)


def _fusion(h11, h12, ls, eps):
    grid = (2, N // BM)
    stk = pl.BlockSpec((1, BM, HALF), lambda c, r: (c, r, 0))
    return pl.pallas_call(
        _fusion_body,
        grid=grid,
        in_specs=[stk, stk, stk,
                  pl.BlockSpec((BM, HALF), lambda c, r: (r, c))],
        out_specs=pl.BlockSpec((BM, HALF), lambda c, r: (r, c)),
        out_shape=jax.ShapeDtypeStruct((N, H), jnp.bfloat16),
    )(h11, h12, ls, eps)


def _decoder_body(zl_ref, zr_ref, o_ref):
    o_ref[...] = lax.dot_general(
        zl_ref[...], zr_ref[...], (((1,), (1,)), ((), ())),
        preferred_element_type=jnp.float32)


def _decoder(z):
    return pl.pallas_call(
        _decoder_body,
        grid=(N // DM,),
        in_specs=[
            pl.BlockSpec((DM, H), lambda r: (r, 0)),
            pl.BlockSpec((N, H), lambda r: (0, 0)),
        ],
        out_specs=pl.BlockSpec((DM, N), lambda r: (r, 0)),
        out_shape=jax.ShapeDtypeStruct((N, N), jnp.float32),
    )(z, z)


def kernel(features, edge_index1, edge_index2, W1_1, W1_2, W1_ls, W2_1, W2_2, W2_ls):
    src1, dst1 = edge_index1[0], edge_index1[1]
    src2, dst2 = edge_index2[0], edge_index2[1]
    zeros = jnp.zeros((RPT_LAST, HALF), jnp.float32)

    xw1, xw2 = _input_matmuls(features, W1_1, W2_1)
    h11, h21 = _enc1(xw1.reshape(2 * N, HALF), src1, dst1,
                     xw2.reshape(2 * N, HALF), src2, dst2, zeros)

    p12, p1ls, p2ls = _hidden_matmuls(
        h11.reshape(2, N, HALF), h21.reshape(2, N, HALF), W1_2, W1_ls, W2_ls)
    h12, ls = _enc2(p12.reshape(2 * N, HALF), src1, dst1,
                    p1ls.reshape(2 * N, HALF), p2ls.reshape(2 * N, HALF),
                    src2, dst2, zeros)

    eps = jax.random.normal(jax.random.key(42), (N, H), dtype=jnp.float32)
    z = _fusion(h11.reshape(2, N, HALF), h12.reshape(2, N, HALF),
                ls.reshape(2, N, HALF), eps)

    rec = _decoder(z).reshape(-1)
    return (rec, rec)
